# baseline (speedup 1.0000x reference)
import functools

import jax
import jax.numpy as jnp
from jax import lax
from jax.experimental import pallas as pl
from jax.experimental.pallas import tpu as pltpu
from jax.experimental.pallas import tpu_sc as plsc

BATCH = 1024
SEQ = 200
EMBED = 128
N = BATCH * SEQ
NW = 32
PER_W = N // NW
CHUNK = 80
NCH = PER_W // CHUNK
K = 10
H = K // 2


def _emb(idx_hbm, table_hbm, out_hbm, idx_v, rows_v, gsem, wsem):
    nc = 2
    wid = lax.axis_index("s") * nc + lax.axis_index("c")
    base = wid * PER_W
    pltpu.sync_copy(idx_hbm.at[pl.ds(base, PER_W)], idx_v)

    def gather(j, b):
        return pltpu.async_copy(
            table_hbm.at[idx_v.at[pl.ds(j * CHUNK, CHUNK)]],
            rows_v.at[pl.ds(b * CHUNK, CHUNK)], gsem.at[b])

    def wait_gather(b):
        pltpu.make_async_copy(
            table_hbm.at[idx_v.at[pl.ds(0, CHUNK)]],
            rows_v.at[pl.ds(b * CHUNK, CHUNK)], gsem.at[b]).wait()

    def write(j, b):
        return pltpu.async_copy(
            rows_v.at[pl.ds(b * CHUNK, CHUNK)],
            out_hbm.at[pl.ds(base + j * CHUNK, CHUNK)], wsem.at[b])

    def wait_write(b):
        pltpu.make_async_copy(
            rows_v.at[pl.ds(b * CHUNK, CHUNK)],
            out_hbm.at[pl.ds(base, CHUNK)], wsem.at[b]).wait()

    for b in range(K):
        gather(b, b)

    def body(j, carry):
        b = lax.rem(j, K)
        wait_gather(b)
        write(j, b)
        r = j - H
        rb = lax.rem(j + H, K)

        @pl.when((r >= 0) & (r + K < NCH))
        def _():
            wait_write(rb)
            gather(r + K, rb)

        return carry

    lax.fori_loop(0, NCH, body, 0)

    def drain(b, carry):
        wait_write(b)
        return carry

    lax.fori_loop(0, K, drain, 0)


@jax.jit
def _lookup(idx, table):
    mesh = plsc.VectorSubcoreMesh(core_axis_name="c", subcore_axis_name="s")
    return pl.kernel(
        _emb,
        out_type=jax.ShapeDtypeStruct((N, EMBED), jnp.float32),
        mesh=mesh,
        scratch_types=[
            pltpu.VMEM((PER_W,), jnp.int32),
            pltpu.VMEM((K * CHUNK, EMBED), jnp.float32),
            pltpu.SemaphoreType.DMA((K,)),
            pltpu.SemaphoreType.DMA((K,)),
        ],
    )(idx, table)


def kernel(x, hidden_h, hidden_c, table):
    idx = x.reshape(N)
    embeds = _lookup(idx, table).reshape(BATCH, SEQ, EMBED)
    return (embeds, hidden_h, hidden_c)


# speedup vs baseline: 1.0072x; 1.0072x over previous
import functools

import jax
import jax.numpy as jnp
from jax import lax
from jax.experimental import pallas as pl
from jax.experimental.pallas import tpu as pltpu
from jax.experimental.pallas import tpu_sc as plsc

BATCH = 1024
SEQ = 200
EMBED = 128
N = BATCH * SEQ
NW = 32
PER_W = N // NW
CHUNK = 64
NCH = PER_W // CHUNK
K = 12
H = K // 2


def _emb(idx_hbm, table_hbm, out_hbm, idx_v, rows_v, gsem, wsem):
    nc = 2
    wid = lax.axis_index("s") * nc + lax.axis_index("c")
    base = wid * PER_W
    pltpu.sync_copy(idx_hbm.at[pl.ds(base, PER_W)], idx_v)

    def gather(j, b):
        return pltpu.async_copy(
            table_hbm.at[idx_v.at[pl.ds(j * CHUNK, CHUNK)]],
            rows_v.at[pl.ds(b * CHUNK, CHUNK)], gsem.at[b])

    def wait_gather(b):
        pltpu.make_async_copy(
            table_hbm.at[idx_v.at[pl.ds(0, CHUNK)]],
            rows_v.at[pl.ds(b * CHUNK, CHUNK)], gsem.at[b]).wait()

    def write(j, b):
        return pltpu.async_copy(
            rows_v.at[pl.ds(b * CHUNK, CHUNK)],
            out_hbm.at[pl.ds(base + j * CHUNK, CHUNK)], wsem.at[b])

    def wait_write(b):
        pltpu.make_async_copy(
            rows_v.at[pl.ds(b * CHUNK, CHUNK)],
            out_hbm.at[pl.ds(base, CHUNK)], wsem.at[b]).wait()

    for b in range(K):
        gather(b, b)

    def body(j, carry):
        b = lax.rem(j, K)
        wait_gather(b)
        write(j, b)
        r = j - H
        rb = lax.rem(j + H, K)

        @pl.when((r >= 0) & (r + K < NCH))
        def _():
            wait_write(rb)
            gather(r + K, rb)

        return carry

    lax.fori_loop(0, NCH, body, 0)

    def drain(b, carry):
        wait_write(b)
        return carry

    lax.fori_loop(0, K, drain, 0)


@jax.jit
def _lookup(idx, table):
    mesh = plsc.VectorSubcoreMesh(core_axis_name="c", subcore_axis_name="s")
    return pl.kernel(
        _emb,
        out_type=jax.ShapeDtypeStruct((N, EMBED), jnp.float32),
        mesh=mesh,
        scratch_types=[
            pltpu.VMEM((PER_W,), jnp.int32),
            pltpu.VMEM((K * CHUNK, EMBED), jnp.float32),
            pltpu.SemaphoreType.DMA((K,)),
            pltpu.SemaphoreType.DMA((K,)),
        ],
    )(idx, table)


def kernel(x, hidden_h, hidden_c, table):
    idx = x.reshape(N)
    embeds = _lookup(idx, table).reshape(BATCH, SEQ, EMBED)
    return (embeds, hidden_h, hidden_c)
